# baseline (speedup 1.0000x reference)
import functools

import jax
import jax.numpy as jnp
from jax.experimental import pallas as pl
from jax.experimental.pallas import tpu as pltpu


def _fused_body(x_ref, adja_ref, adjb_ref, w1_ref, b1_ref, w2_ref, b2_ref,
                outa_ref, outb_ref, *, tm, half):
    i = pl.program_id(0)

    def panel(adj_ref, out_ref, row0):
        agg = jnp.dot(adj_ref[...], x_ref[...],
                      preferred_element_type=jnp.float32)
        h = agg + x_ref[pl.ds(row0, tm), :]
        h = jnp.maximum(
            jnp.dot(h, w1_ref[...], preferred_element_type=jnp.float32)
            + b1_ref[...], 0.0)
        out_ref[...] = (jnp.dot(h, w2_ref[...],
                                preferred_element_type=jnp.float32)
                        + b2_ref[...])

    panel(adja_ref, outa_ref, i * tm)
    panel(adjb_ref, outb_ref, half + i * tm)


@jax.jit
def _run(x2, adj, W1, b1r, W2, b2r):
    n, d = x2.shape
    tm = 200
    half = n // 2
    nb = half // tm
    outa, outb = pl.pallas_call(
        functools.partial(_fused_body, tm=tm, half=half),
        grid=(nb,),
        in_specs=[
            pl.BlockSpec((n, d), lambda i: (0, 0)),
            pl.BlockSpec((tm, n), lambda i: (i, 0)),
            pl.BlockSpec((tm, n), lambda i: (i + 25, 0)),
            pl.BlockSpec((d, d), lambda i: (0, 0)),
            pl.BlockSpec((1, d), lambda i: (0, 0)),
            pl.BlockSpec((d, d), lambda i: (0, 0)),
            pl.BlockSpec((1, d), lambda i: (0, 0)),
        ],
        out_specs=[
            pl.BlockSpec((tm, d), lambda i: (i, 0)),
            pl.BlockSpec((tm, d), lambda i: (i, 0)),
        ],
        out_shape=[
            jax.ShapeDtypeStruct((half, d), jnp.float32),
            jax.ShapeDtypeStruct((half, d), jnp.float32),
        ],
        compiler_params=pltpu.CompilerParams(
            dimension_semantics=("arbitrary",),
        ),
    )(x2, adj, adj, W1, b1r, W2, b2r)
    return jnp.concatenate([outa, outb], axis=0)


def kernel(x, adj, W1, b1, W2, b2):
    if adj.ndim == 3:
        adj = adj[0]
    x2 = x[0]
    out = _run(x2, adj, W1, b1.reshape(1, -1), W2, b2.reshape(1, -1))
    return out[None]
